# Initial kernel scaffold; baseline (speedup 1.0000x reference)
import jax
import jax.numpy as jnp
from jax.experimental import pallas as pl


def kernel(x23, indices, emb_table, gamma, beta):
    raise NotImplementedError("write your pallas kernel here")


# speedup vs baseline: 2.1365x; 2.1365x over previous
import functools

import jax
import jax.numpy as jnp
from jax.experimental import pallas as pl
from jax.experimental.pallas import tpu as pltpu

_B = 256
_S = 384
_D = 128
_V = 512
_BT = 32


def _ln_kernel(idx_ref, emb_ref, x_ref, gamma_ref, beta_ref, out_ref, x25_ref):
    i = pl.program_id(0)

    @pl.when(i == 0)
    def _():
        idx = idx_ref[...]
        row_ids = jax.lax.broadcasted_iota(jnp.int32, (_V, _S), 0)
        onehot_t = (row_ids == idx).astype(jnp.float32)
        x25_ref[...] = jax.lax.dot_general(
            onehot_t,
            emb_ref[...],
            (((0,), (0,)), ((), ())),
            preferred_element_type=jnp.float32,
        )

    x = x_ref[...] + x25_ref[...][None, :, :]
    mean = jnp.mean(x, axis=-1, keepdims=True)
    xc = x - mean
    var = jnp.mean(xc * xc, axis=-1, keepdims=True)
    normed = xc * jax.lax.rsqrt(var + 1e-12)
    gamma = gamma_ref[...].reshape(1, 1, _D)
    beta = beta_ref[...].reshape(1, 1, _D)
    out_ref[...] = normed * gamma + beta


@jax.jit
def kernel(x23, indices, emb_table, gamma, beta):
    idx32 = indices.astype(jnp.int32).reshape(1, _S)
    grid = (_B // _BT,)
    return pl.pallas_call(
        _ln_kernel,
        grid=grid,
        in_specs=[
            pl.BlockSpec((1, _S), lambda i: (0, 0)),
            pl.BlockSpec((_V, _D), lambda i: (0, 0)),
            pl.BlockSpec((_BT, _S, _D), lambda i: (i, 0, 0)),
            pl.BlockSpec((1, _D), lambda i: (0, 0)),
            pl.BlockSpec((1, _D), lambda i: (0, 0)),
        ],
        out_specs=pl.BlockSpec((_BT, _S, _D), lambda i: (i, 0, 0)),
        out_shape=jax.ShapeDtypeStruct((_B, _S, _D), jnp.float32),
        scratch_shapes=[pltpu.VMEM((_S, _D), jnp.float32)],
    )(idx32, emb_table, x23, gamma.reshape(1, _D), beta.reshape(1, _D))
